# baseline (speedup 1.0000x reference)
import functools

import jax
import jax.numpy as jnp
from jax import lax
from jax.experimental import pallas as pl
from jax.experimental.pallas import tpu as pltpu
from jax.experimental.pallas import tpu_sc as plsc

_NUM_WORKERS = 32
_MLP_BB = 256


def _sc_gather(codebook, idx):
    b_total = idx.shape[0]
    d = codebook.shape[1]
    b_per_w = b_total // _NUM_WORKERS
    mesh = plsc.VectorSubcoreMesh(core_axis_name="c", subcore_axis_name="s")

    @functools.partial(
        pl.kernel,
        mesh=mesh,
        out_type=jax.ShapeDtypeStruct((b_total, d), jnp.float32),
        scratch_types=[
            pltpu.VMEM((b_per_w,), jnp.int32),
            pltpu.VMEM((b_per_w, d), jnp.float32),
            pltpu.SemaphoreType.DMA,
        ],
        compiler_params=pltpu.CompilerParams(use_tc_tiling_on_sc=False),
    )
    def gather_kernel(table_hbm, idx_hbm, out_hbm, idx_v, rows_v, sem):
        wid = lax.axis_index("s") * 2 + lax.axis_index("c")
        base = wid * b_per_w
        pltpu.sync_copy(idx_hbm.at[pl.ds(base, b_per_w)], idx_v)
        pltpu.async_copy(table_hbm.at[idx_v], rows_v, sem).wait()
        pltpu.sync_copy(rows_v, out_hbm.at[pl.ds(base, b_per_w)])

    return gather_kernel(codebook, idx)


def _mlp_body(alpha_ref, w1_ref, b1_ref, w2_ref, b2_ref, w3_ref, b3_ref, out_ref):
    h = jnp.dot(alpha_ref[...], w1_ref[...], preferred_element_type=jnp.float32)
    h = jax.nn.gelu(h + b1_ref[...])
    h = jnp.dot(h, w2_ref[...], preferred_element_type=jnp.float32)
    h = jax.nn.gelu(h + b2_ref[...])
    out = jnp.dot(h, w3_ref[...], preferred_element_type=jnp.float32)
    out_ref[...] = out + b3_ref[...]


def _mlp(alpha, W1, b1, W2, b2, W3, b3):
    b_total, d = alpha.shape
    h1 = W1.shape[1]
    h2 = W2.shape[1]
    c = W3.shape[1]
    grid = (b_total // _MLP_BB,)
    return pl.pallas_call(
        _mlp_body,
        grid=grid,
        in_specs=[
            pl.BlockSpec((_MLP_BB, d), lambda i: (i, 0)),
            pl.BlockSpec((d, h1), lambda i: (0, 0)),
            pl.BlockSpec((1, h1), lambda i: (0, 0)),
            pl.BlockSpec((h1, h2), lambda i: (0, 0)),
            pl.BlockSpec((1, h2), lambda i: (0, 0)),
            pl.BlockSpec((h2, c), lambda i: (0, 0)),
            pl.BlockSpec((1, c), lambda i: (0, 0)),
        ],
        out_specs=pl.BlockSpec((_MLP_BB, c), lambda i: (i, 0)),
        out_shape=jax.ShapeDtypeStruct((b_total, c), jnp.float32),
        compiler_params=pltpu.CompilerParams(
            dimension_semantics=("parallel",),
        ),
    )(alpha, W1, b1.reshape(1, -1), W2, b2.reshape(1, -1), W3, b3.reshape(1, -1))


def kernel(chunk_ids, codebook, W1, b1, W2, b2, W3, b3):
    alpha = _sc_gather(codebook, chunk_ids.astype(jnp.int32))
    return _mlp(alpha, W1, b1, W2, b2, W3, b3)


# speedup vs baseline: 1.6773x; 1.6773x over previous
import functools

import jax
import jax.numpy as jnp
from jax import lax
from jax.experimental import pallas as pl
from jax.experimental.pallas import tpu as pltpu
from jax.experimental.pallas import tpu_sc as plsc

_NUM_WORKERS = 32
_LANES = 16
_MLP_BB = 256


def _sc_gather(codebook, idx):
    b_total = idx.shape[0]
    d = codebook.shape[1]
    b_per_w = b_total // _NUM_WORKERS
    n_chunks16 = b_per_w // _LANES
    mesh = plsc.VectorSubcoreMesh(core_axis_name="c", subcore_axis_name="s")

    @functools.partial(
        pl.kernel,
        mesh=mesh,
        out_type=jax.ShapeDtypeStruct((b_total, d), jnp.float32),
        scratch_types=[
            pltpu.VMEM((b_per_w,), jnp.int32),
            pltpu.VMEM((b_per_w, d), jnp.float32),
            pltpu.SemaphoreType.DMA,
        ],
        compiler_params=pltpu.CompilerParams(needs_layout_passes=False),
    )
    def gather_kernel(table_hbm, idx_hbm, out_hbm, idx_v, alpha_v, sem):
        wid = lax.axis_index("s") * 2 + lax.axis_index("c")
        base = wid * b_per_w
        pltpu.sync_copy(idx_hbm.at[pl.ds(base, b_per_w)], idx_v)

        for k in range(n_chunks16):
            ids = idx_v[pl.ds(k * _LANES, _LANES)]
            for l in range(_LANES):
                j = k * _LANES + l
                pltpu.async_copy(
                    table_hbm.at[pl.ds(ids[l], 1)], alpha_v.at[pl.ds(j, 1)],
                    sem,
                )
        pltpu.make_async_copy(
            table_hbm.at[pl.ds(0, b_per_w)], alpha_v, sem
        ).wait()
        pltpu.sync_copy(alpha_v, out_hbm.at[pl.ds(base, b_per_w)])

    return gather_kernel(codebook, idx)


def _mlp_body(alpha_ref, w1_ref, b1_ref, w2_ref, b2_ref, w3_ref, b3_ref, out_ref):
    h = jnp.dot(alpha_ref[...], w1_ref[...], preferred_element_type=jnp.float32)
    h = jax.nn.gelu(h + b1_ref[...])
    h = jnp.dot(h, w2_ref[...], preferred_element_type=jnp.float32)
    h = jax.nn.gelu(h + b2_ref[...])
    out = jnp.dot(h, w3_ref[...], preferred_element_type=jnp.float32)
    out_ref[...] = out + b3_ref[...]


def _mlp(alpha, W1, b1, W2, b2, W3, b3):
    b_total, d = alpha.shape
    h1 = W1.shape[1]
    h2 = W2.shape[1]
    c = W3.shape[1]
    grid = (b_total // _MLP_BB,)
    return pl.pallas_call(
        _mlp_body,
        grid=grid,
        in_specs=[
            pl.BlockSpec((_MLP_BB, d), lambda i: (i, 0)),
            pl.BlockSpec((d, h1), lambda i: (0, 0)),
            pl.BlockSpec((1, h1), lambda i: (0, 0)),
            pl.BlockSpec((h1, h2), lambda i: (0, 0)),
            pl.BlockSpec((1, h2), lambda i: (0, 0)),
            pl.BlockSpec((h2, c), lambda i: (0, 0)),
            pl.BlockSpec((1, c), lambda i: (0, 0)),
        ],
        out_specs=pl.BlockSpec((_MLP_BB, c), lambda i: (i, 0)),
        out_shape=jax.ShapeDtypeStruct((b_total, c), jnp.float32),
        compiler_params=pltpu.CompilerParams(
            dimension_semantics=("parallel",),
        ),
    )(alpha, W1, b1.reshape(1, -1), W2, b2.reshape(1, -1), W3, b3.reshape(1, -1))


def kernel(chunk_ids, codebook, W1, b1, W2, b2, W3, b3):
    alpha = _sc_gather(codebook, chunk_ids.astype(jnp.int32))
    return _mlp(alpha, W1, b1, W2, b2, W3, b3)


# speedup vs baseline: 2.4280x; 1.4476x over previous
import functools

import jax
import jax.numpy as jnp
from jax import lax
from jax.experimental import pallas as pl
from jax.experimental.pallas import tpu as pltpu
from jax.experimental.pallas import tpu_sc as plsc

_NUM_WORKERS = 32
_LANES = 16
_MLP_BB = 256


def _sc_gather(codebook, idx):
    b_total = idx.shape[0]
    d = codebook.shape[1]
    b_per_w = b_total // _NUM_WORKERS
    n_chunks16 = b_per_w // _LANES
    mesh = plsc.VectorSubcoreMesh(core_axis_name="c", subcore_axis_name="s")

    @functools.partial(
        pl.kernel,
        mesh=mesh,
        out_type=jax.ShapeDtypeStruct((b_total, d), jnp.float32),
        scratch_types=[
            pltpu.VMEM((b_per_w,), jnp.int32),
            pltpu.VMEM((b_per_w, d), jnp.float32),
            pltpu.SemaphoreType.DMA,
        ],
        compiler_params=pltpu.CompilerParams(needs_layout_passes=False),
    )
    def gather_kernel(table_hbm, idx_hbm, out_hbm, idx_v, alpha_v, sem):
        wid = lax.axis_index("s") * 2 + lax.axis_index("c")
        base = wid * b_per_w
        pltpu.sync_copy(idx_hbm.at[pl.ds(base, b_per_w)], idx_v)

        for k in range(n_chunks16):
            ids = idx_v[pl.ds(k * _LANES, _LANES)]
            for l in range(_LANES):
                j = k * _LANES + l
                pltpu.async_copy(
                    table_hbm.at[pl.ds(ids[l], 1)], alpha_v.at[pl.ds(j, 1)],
                    sem,
                )
        pltpu.make_async_copy(
            table_hbm.at[pl.ds(0, b_per_w)], alpha_v, sem
        ).wait()
        pltpu.sync_copy(alpha_v, out_hbm.at[pl.ds(base, b_per_w)])

    return gather_kernel(codebook, idx)


def _mlp_body(alpha_ref, w1_ref, b1_ref, w2_ref, b2_ref, w3_ref, b3_ref, out_ref):
    h = jnp.dot(alpha_ref[...], w1_ref[...], preferred_element_type=jnp.float32)
    h = jax.nn.gelu(h + b1_ref[...])
    h = jnp.dot(h, w2_ref[...], preferred_element_type=jnp.float32)
    h = jax.nn.gelu(h + b2_ref[...])
    out = jnp.dot(h, w3_ref[...], preferred_element_type=jnp.float32)
    out_ref[...] = out + b3_ref[...]


def _mlp(alpha, W1, b1, W2, b2, W3, b3):
    b_total, d = alpha.shape
    h1 = W1.shape[1]
    h2 = W2.shape[1]
    c = W3.shape[1]
    grid = (b_total // _MLP_BB,)
    return pl.pallas_call(
        _mlp_body,
        grid=grid,
        in_specs=[
            pl.BlockSpec((_MLP_BB, d), lambda i: (i, 0)),
            pl.BlockSpec((d, h1), lambda i: (0, 0)),
            pl.BlockSpec((1, h1), lambda i: (0, 0)),
            pl.BlockSpec((h1, h2), lambda i: (0, 0)),
            pl.BlockSpec((1, h2), lambda i: (0, 0)),
            pl.BlockSpec((h2, c), lambda i: (0, 0)),
            pl.BlockSpec((1, c), lambda i: (0, 0)),
        ],
        out_specs=pl.BlockSpec((_MLP_BB, c), lambda i: (i, 0)),
        out_shape=jax.ShapeDtypeStruct((b_total, c), jnp.float32),
        compiler_params=pltpu.CompilerParams(
            dimension_semantics=("parallel",),
        ),
    )(alpha, W1, b1.reshape(1, -1), W2, b2.reshape(1, -1), W3, b3.reshape(1, -1))


def kernel(chunk_ids, codebook, W1, b1, W2, b2, W3, b3):
    alpha = jnp.take(codebook, chunk_ids, axis=0)
    return _mlp(alpha, W1, b1, W2, b2, W3, b3)
